# baseline (speedup 1.0000x reference)
import functools

import jax
import jax.numpy as jnp
from jax import lax
from jax.experimental import pallas as pl
from jax.experimental.pallas import tpu as pltpu
from jax.experimental.pallas import tpu_sc as plsc

N = 10000
E = 320000
F_IN = 128
H = 256
D_E = 16
L = 3
G = 64

NPAD = 10240
NB = 1000
EB = 2560
K = 80
NCORES = 2
NSUB = 16
NWORK = NCORES * NSUB
HALF0 = 161280
HALF1 = E - HALF0


def _pack_tables(td):
    u = lax.bitcast_convert_type(td, jnp.uint32)
    rne = lambda w: ((w + jnp.uint32(0x7FFF) + ((w >> 16) & jnp.uint32(1)))
                     & jnp.uint32(0xFFFF0000))
    return (rne(u[:, :H]) >> 16) | rne(u[:, H:])


def _unpack_tables(w):
    f = lax.bitcast_convert_type(w << 16, jnp.float32)
    s = lax.bitcast_convert_type(w & jnp.uint32(0xFFFF0000), jnp.float32)
    return f, s

def _emb_tables_body(x_ref, we_ref, be_ref, wd_ref, ws_ref,
                     h_ref, td_ref, ts_ref):
    h = jnp.dot(x_ref[...], we_ref[...],
                preferred_element_type=jnp.float32) + be_ref[...]
    h_ref[...] = h
    td = jnp.dot(h, wd_ref[...], preferred_element_type=jnp.float32)
    ts = jnp.dot(h, ws_ref[...], preferred_element_type=jnp.float32)
    td_ref[...] = _pack_tables(td)
    ts_ref[...] = _pack_tables(ts)


def _emb_tables(x, w_emb, b_emb, wd, wsr):
    return pl.pallas_call(
        _emb_tables_body,
        grid=(N // NB,),
        in_specs=[
            pl.BlockSpec((NB, F_IN), lambda i: (i, 0)),
            pl.BlockSpec((F_IN, H), lambda i: (0, 0)),
            pl.BlockSpec((1, H), lambda i: (0, 0)),
            pl.BlockSpec((H, 2 * H), lambda i: (0, 0)),
            pl.BlockSpec((H, 2 * H), lambda i: (0, 0)),
        ],
        out_specs=[
            pl.BlockSpec((NB, H), lambda i: (i, 0)),
            pl.BlockSpec((NB, H), lambda i: (i, 0)),
            pl.BlockSpec((NB, H), lambda i: (i, 0)),
        ],
        out_shape=[
            jax.ShapeDtypeStruct((N, H), jnp.float32),
            jax.ShapeDtypeStruct((N, H), jnp.uint32),
            jax.ShapeDtypeStruct((N, H), jnp.uint32),
        ],
    )(x, w_emb, b_emb, wd, wsr)


def _edge_msg_body(fsd_ref, fss_ref, ea_ref, we_ref, b_ref, m_ref):
    c = jnp.dot(ea_ref[...], we_ref[...],
                preferred_element_type=jnp.float32) + b_ref[...]
    fd, sd = _unpack_tables(fsd_ref[...])
    fs_, ss_ = _unpack_tables(fss_ref[...])
    f = fd + fs_ + c[:, :H]
    s = sd + ss_ + c[:, H:]
    m = jax.nn.sigmoid(f) * jax.nn.softplus(s)
    m_ref[0] = m[:, :H // 2]
    m_ref[1] = m[:, H // 2:]


def _edge_msg(fsd, fss, ea, we, b, e_off, cnt):
    ob = e_off // EB
    return pl.pallas_call(
        _edge_msg_body,
        grid=(cnt // EB,),
        in_specs=[
            pl.BlockSpec((EB, H), lambda i: (i, 0)),
            pl.BlockSpec((EB, H), lambda i: (i, 0)),
            pl.BlockSpec((EB, D_E), lambda i: (i + ob, 0)),
            pl.BlockSpec((D_E, 2 * H), lambda i: (0, 0)),
            pl.BlockSpec((1, 2 * H), lambda i: (0, 0)),
        ],
        out_specs=pl.BlockSpec((2, EB, H // 2), lambda i: (0, i, 0)),
        out_shape=jax.ShapeDtypeStruct((2, cnt, H // 2), jnp.float32),
    )(fsd, fss, ea, we, b)


def _stats_body(h_ref, agg_ref, agg1_ref, r_ref, st_ref):
    agg = (jnp.concatenate([agg_ref[0], agg_ref[1]], axis=1) +
           jnp.concatenate([agg1_ref[0], agg1_ref[1]], axis=1))
    r = jnp.maximum(h_ref[...] + agg, 0.0)
    r_ref[...] = r

    @pl.when(pl.program_id(0) == 0)
    def _():
        st_ref[...] = jnp.zeros_like(st_ref)

    s = jnp.sum(r, axis=0)[None]
    s2 = jnp.sum(r * r, axis=0)[None]
    pad = jnp.zeros((6, H), jnp.float32)
    st_ref[...] = st_ref[...] + jnp.concatenate([s, s2, pad], axis=0)


def _stats(h, agg, agg1):
    return pl.pallas_call(
        _stats_body,
        grid=(N // NB,),
        in_specs=[
            pl.BlockSpec((NB, H), lambda i: (i, 0)),
            pl.BlockSpec((2, NB, H // 2), lambda i: (0, i, 0)),
            pl.BlockSpec((2, NB, H // 2), lambda i: (0, i, 0)),
        ],
        out_specs=[
            pl.BlockSpec((NB, H), lambda i: (i, 0)),
            pl.BlockSpec((8, H), lambda i: (0, 0)),
        ],
        out_shape=[
            jax.ShapeDtypeStruct((N, H), jnp.float32),
            jax.ShapeDtypeStruct((8, H), jnp.float32),
        ],
    )(h, agg, agg1)


def _bn(r, st_ref, g_ref, b_ref):
    mu = st_ref[0:1, :] / N
    var = st_ref[1:2, :] / N - mu * mu
    rstd = lax.rsqrt(var + 1e-5)
    return (r - mu) * rstd * g_ref[...] + b_ref[...]


def _norm_tables_body(r_ref, st_ref, g_ref, b_ref, wd_ref, ws_ref,
                      h_ref, td_ref, ts_ref):
    hn = _bn(r_ref[...], st_ref, g_ref, b_ref)
    h_ref[...] = hn
    td = jnp.dot(hn, wd_ref[...], preferred_element_type=jnp.float32)
    ts = jnp.dot(hn, ws_ref[...], preferred_element_type=jnp.float32)
    td_ref[...] = _pack_tables(td)
    ts_ref[...] = _pack_tables(ts)


def _norm_tables(r, st, g, b, wd, wsr):
    return pl.pallas_call(
        _norm_tables_body,
        grid=(N // NB,),
        in_specs=[
            pl.BlockSpec((NB, H), lambda i: (i, 0)),
            pl.BlockSpec((8, H), lambda i: (0, 0)),
            pl.BlockSpec((1, H), lambda i: (0, 0)),
            pl.BlockSpec((1, H), lambda i: (0, 0)),
            pl.BlockSpec((H, 2 * H), lambda i: (0, 0)),
            pl.BlockSpec((H, 2 * H), lambda i: (0, 0)),
        ],
        out_specs=[
            pl.BlockSpec((NB, H), lambda i: (i, 0)),
            pl.BlockSpec((NB, H), lambda i: (i, 0)),
            pl.BlockSpec((NB, H), lambda i: (i, 0)),
        ],
        out_shape=[
            jax.ShapeDtypeStruct((N, H), jnp.float32),
            jax.ShapeDtypeStruct((N, H), jnp.uint32),
            jax.ShapeDtypeStruct((N, H), jnp.uint32),
        ],
    )(r, st, g, b, wd, wsr)


def _final_body(r_ref, st_ref, g_ref, b_ref, bat_ref, w1_ref, b1_ref,
                w2_ref, b2_ref, out_ref):
    hn = _bn(r_ref[...], st_ref, g_ref, b_ref)
    gids = lax.broadcasted_iota(jnp.int32, (1, G), 1)
    onehot = (bat_ref[...] == gids).astype(jnp.float32)
    sums = lax.dot_general(onehot, hn, (((0,), (0,)), ((), ())),
                           preferred_element_type=jnp.float32)
    counts = jnp.sum(onehot, axis=0)[:, None]
    gemb = sums / jnp.maximum(counts, 1.0)
    o = jnp.maximum(jnp.dot(gemb, w1_ref[...],
                            preferred_element_type=jnp.float32) + b1_ref[...],
                    0.0)
    out_ref[...] = jnp.dot(o, w2_ref[...],
                           preferred_element_type=jnp.float32) + b2_ref[...]


def _final(r, st, g, b, batch2d, w1, b1, w2, b2):
    return pl.pallas_call(
        _final_body,
        out_shape=jax.ShapeDtypeStruct((G, 1), jnp.float32),
    )(r, st, g, b, batch2d, w1, b1, w2, b2)


KG = 40


def _sc_gather2(td, tsrc, dst, src, e_off, cnt):
    ep = cnt // NWORK
    nch = ep // KG
    nit = nch // 2
    mesh = plsc.VectorSubcoreMesh(core_axis_name="c", subcore_axis_name="s")

    @functools.partial(
        pl.kernel, mesh=mesh,
        out_type=[jax.ShapeDtypeStruct((cnt, H), jnp.uint32),
                  jax.ShapeDtypeStruct((cnt, H), jnp.uint32)],
        scratch_types=[
            pltpu.VMEM((KG,), jnp.int32),
            pltpu.VMEM((KG,), jnp.int32),
            pltpu.VMEM((KG,), jnp.int32),
            pltpu.VMEM((KG,), jnp.int32),
            pltpu.VMEM((KG, H), jnp.uint32),
            pltpu.VMEM((KG, H), jnp.uint32),
            pltpu.VMEM((KG, H), jnp.uint32),
            pltpu.VMEM((KG, H), jnp.uint32),
            pltpu.SemaphoreType.DMA,
            pltpu.SemaphoreType.DMA,
            pltpu.SemaphoreType.DMA,
            pltpu.SemaphoreType.DMA,
            pltpu.SemaphoreType.DMA,
        ],
    )
    def k(td_h, ts_h, dst_h, src_h, fsd_h, fss_h,
          di0, si0, di1, si1, rd0, rs0, rd1, rs1, sd0, ss0, sd1, ss1, sw):
        wid = lax.axis_index("s") * NCORES + lax.axis_index("c")
        base = wid * ep

        def gath(j, di, si, rd, rs, s_d, s_s):
            e0 = base + j * KG
            pltpu.sync_copy(dst_h.at[pl.ds(e0 + e_off, KG)], di)
            pltpu.sync_copy(src_h.at[pl.ds(e0 + e_off, KG)], si)
            pltpu.async_copy(td_h.at[di], rd, s_d)
            pltpu.async_copy(ts_h.at[si], rs, s_s)

        def waitg(di, si, rd, rs, s_d, s_s):
            pltpu.make_async_copy(td_h.at[di], rd, s_d).wait()
            pltpu.make_async_copy(ts_h.at[si], rs, s_s).wait()

        def put(j, rd, rs):
            pltpu.async_copy(rd, fsd_h.at[pl.ds(base + j * KG, KG)], sw)
            pltpu.async_copy(rs, fss_h.at[pl.ds(base + j * KG, KG)], sw)

        def waitw(rd, rs):
            pltpu.make_async_copy(rd, fsd_h.at[pl.ds(base, KG)], sw).wait()
            pltpu.make_async_copy(rs, fss_h.at[pl.ds(base, KG)], sw).wait()

        gath(0, di0, si0, rd0, rs0, sd0, ss0)

        def body(i, carry):
            j0 = 2 * i
            j1 = j0 + 1

            @pl.when(i > 0)
            def _():
                waitw(rd1, rs1)

            gath(j1, di1, si1, rd1, rs1, sd1, ss1)
            waitg(di0, si0, rd0, rs0, sd0, ss0)
            put(j0, rd0, rs0)

            @pl.when(i < nit - 1)
            def _():
                waitw(rd0, rs0)
                gath(j0 + 2, di0, si0, rd0, rs0, sd0, ss0)

            waitg(di1, si1, rd1, rs1, sd1, ss1)
            put(j1, rd1, rs1)
            return carry

        lax.fori_loop(0, nit, body, 0)
        waitw(rd0, rs0)
        waitw(rd1, rs1)

    return k(td, tsrc, dst, src)


def _sc_scatter_add(m, dst, e_off, cnt, init=None):
    tpc = cnt // NSUB
    nch = tpc // K
    rows_per_tile = NPAD // NSUB
    hc = H // 2
    mesh = plsc.VectorSubcoreMesh(core_axis_name="c", subcore_axis_name="s")

    nit = nch // 2

    @functools.partial(
        pl.kernel, mesh=mesh,
        out_type=jax.ShapeDtypeStruct((2, NPAD, hc), jnp.float32),
        scratch_types=[
            pltpu.VMEM((K,), jnp.int32),
            pltpu.VMEM((K,), jnp.int32),
            pltpu.VMEM((K, hc), jnp.float32),
            pltpu.VMEM((K, hc), jnp.float32),
            pltpu.VMEM((K, hc), jnp.float32),
            pltpu.VMEM_SHARED((NPAD, hc), jnp.float32),
            pltpu.SemaphoreType.DMA,
            pltpu.SemaphoreType.DMA,
        ],
    )
    def k(m_h, dst_h, *args):
        if init is None:
            agg_h, di0, di1, mv0, mv1, zv, acc, s0, s1 = args
        else:
            init_h, agg_h, di0, di1, mv0, mv1, zv, acc, s0, s1 = args
        c = lax.axis_index("c")
        s = lax.axis_index("s")
        r0 = s * rows_per_tile

        if init is None:
            def zrow(e, carry):
                for cc in range(hc // 16):
                    zv[e, pl.ds(cc * 16, 16)] = jnp.zeros((16,), jnp.float32)
                return carry

            lax.fori_loop(0, K, zrow, 0)

            def zchunk(j, carry):
                pltpu.sync_copy(zv, acc.at[pl.ds(r0 + j * K, K)])
                return carry

            lax.fori_loop(0, rows_per_tile // K, zchunk, 0)
        else:
            pltpu.sync_copy(init_h.at[c, pl.ds(r0, rows_per_tile)],
                            acc.at[pl.ds(r0, rows_per_tile)])

        plsc.subcore_barrier()

        base = s * tpc

        def getm(j, di, mv, sem):
            e0 = base + j * K
            pltpu.sync_copy(dst_h.at[pl.ds(e0 + e_off, K)], di)
            pltpu.async_copy(m_h.at[c, pl.ds(e0, K)], mv, sem)

        def waitm(mv, sem):
            pltpu.make_async_copy(m_h.at[c, pl.ds(base, K)], mv, sem).wait()

        getm(0, di0, mv0, s0)

        def body(i, carry):
            j0 = 2 * i
            j1 = j0 + 1
            getm(j1, di1, mv1, s1)
            waitm(mv0, s0)
            pltpu.sync_copy(mv0, acc.at[di0], add=True)

            @pl.when(i < nit - 1)
            def _():
                getm(j0 + 2, di0, mv0, s0)

            waitm(mv1, s1)
            pltpu.sync_copy(mv1, acc.at[di1], add=True)
            return carry

        lax.fori_loop(0, nit, body, 0)
        plsc.subcore_barrier()
        pltpu.sync_copy(acc.at[pl.ds(r0, rows_per_tile)],
                        agg_h.at[c, pl.ds(r0, rows_per_tile)])

    if init is None:
        return k(m, dst)
    return k(m, dst, init)


def kernel(x, edge_index, edge_attr, batch, W_emb, b_emb, Wf, bf, Ws, bs,
           gamma, beta, W1, b1, W2, b2):
    src = edge_index[0]
    dst = edge_index[1]

    wd = [jnp.concatenate([Wf[i, :H], Ws[i, :H]], axis=1) for i in range(L)]
    wsr = [jnp.concatenate([Wf[i, H:2 * H], Ws[i, H:2 * H]], axis=1)
           for i in range(L)]
    we = [jnp.concatenate([Wf[i, 2 * H:], Ws[i, 2 * H:]], axis=1)
          for i in range(L)]
    bfs = [jnp.concatenate([bf[i], bs[i]])[None] for i in range(L)]

    h, td, tsrc = _emb_tables(x, W_emb, b_emb[None], wd[0], wsr[0])

    for i in range(L):
        fsd0, fss0 = _sc_gather2(td, tsrc, dst, src, 0, HALF0)
        m0 = _edge_msg(fsd0, fss0, edge_attr, we[i], bfs[i], 0, HALF0)
        fsd1, fss1 = _sc_gather2(td, tsrc, dst, src, HALF0, HALF1)
        m1 = _edge_msg(fsd1, fss1, edge_attr, we[i], bfs[i], HALF0, HALF1)
        agg0 = _sc_scatter_add(m0, dst, 0, HALF0)
        agg1 = _sc_scatter_add(m1, dst, HALF0, HALF1)
        r, st = _stats(h, agg0[:, :N, :], agg1[:, :N, :])
        if i < L - 1:
            h, td, tsrc = _norm_tables(r, st, gamma[i][None], beta[i][None],
                                       wd[i + 1], wsr[i + 1])
        else:
            pred = _final(r, st, gamma[i][None], beta[i][None],
                          batch[:, None], W1, b1[None], W2, b2[None])
    return pred.reshape((G,))


# speedup vs baseline: 1.0040x; 1.0040x over previous
import functools

import jax
import jax.numpy as jnp
from jax import lax
from jax.experimental import pallas as pl
from jax.experimental.pallas import tpu as pltpu
from jax.experimental.pallas import tpu_sc as plsc

N = 10000
E = 320000
F_IN = 128
H = 256
D_E = 16
L = 3
G = 64

NPAD = 10240
NB = 1000
EB = 2560
K = 80
NCORES = 2
NSUB = 16
NWORK = NCORES * NSUB
HALF0 = 161280
HALF1 = E - HALF0


def _pack_tables(td):
    u = lax.bitcast_convert_type(td, jnp.uint32)
    rne = lambda w: ((w + jnp.uint32(0x7FFF) + ((w >> 16) & jnp.uint32(1)))
                     & jnp.uint32(0xFFFF0000))
    return (rne(u[:, :H]) >> 16) | rne(u[:, H:])


def _unpack_tables(w):
    f = lax.bitcast_convert_type(w << 16, jnp.float32)
    s = lax.bitcast_convert_type(w & jnp.uint32(0xFFFF0000), jnp.float32)
    return f, s

def _emb_tables_body(x_ref, we_ref, be_ref, wd_ref, ws_ref,
                     h_ref, td_ref, ts_ref):
    h = jnp.dot(x_ref[...], we_ref[...],
                preferred_element_type=jnp.float32) + be_ref[...]
    h_ref[...] = h
    td = jnp.dot(h, wd_ref[...], preferred_element_type=jnp.float32)
    ts = jnp.dot(h, ws_ref[...], preferred_element_type=jnp.float32)
    td_ref[...] = _pack_tables(td)
    ts_ref[...] = _pack_tables(ts)


def _emb_tables(x, w_emb, b_emb, wd, wsr):
    return pl.pallas_call(
        _emb_tables_body,
        grid=(N // NB,),
        in_specs=[
            pl.BlockSpec((NB, F_IN), lambda i: (i, 0)),
            pl.BlockSpec((F_IN, H), lambda i: (0, 0)),
            pl.BlockSpec((1, H), lambda i: (0, 0)),
            pl.BlockSpec((H, 2 * H), lambda i: (0, 0)),
            pl.BlockSpec((H, 2 * H), lambda i: (0, 0)),
        ],
        out_specs=[
            pl.BlockSpec((NB, H), lambda i: (i, 0)),
            pl.BlockSpec((NB, H), lambda i: (i, 0)),
            pl.BlockSpec((NB, H), lambda i: (i, 0)),
        ],
        out_shape=[
            jax.ShapeDtypeStruct((N, H), jnp.float32),
            jax.ShapeDtypeStruct((N, H), jnp.uint32),
            jax.ShapeDtypeStruct((N, H), jnp.uint32),
        ],
    )(x, w_emb, b_emb, wd, wsr)


def _edge_msg_body(fsd_ref, fss_ref, ea_ref, we_ref, b_ref, m_ref):
    c = jnp.dot(ea_ref[...], we_ref[...],
                preferred_element_type=jnp.float32) + b_ref[...]
    fd, sd = _unpack_tables(fsd_ref[...])
    fs_, ss_ = _unpack_tables(fss_ref[...])
    f = fd + fs_ + c[:, :H]
    s = sd + ss_ + c[:, H:]
    m = jax.nn.sigmoid(f) * jax.nn.softplus(s)
    m_ref[0] = m[:, :H // 2]
    m_ref[1] = m[:, H // 2:]


def _edge_msg(fsd, fss, ea, we, b, e_off, cnt):
    ob = e_off // EB
    return pl.pallas_call(
        _edge_msg_body,
        grid=(cnt // EB,),
        in_specs=[
            pl.BlockSpec((EB, H), lambda i: (i, 0)),
            pl.BlockSpec((EB, H), lambda i: (i, 0)),
            pl.BlockSpec((EB, D_E), lambda i: (i + ob, 0)),
            pl.BlockSpec((D_E, 2 * H), lambda i: (0, 0)),
            pl.BlockSpec((1, 2 * H), lambda i: (0, 0)),
        ],
        out_specs=pl.BlockSpec((2, EB, H // 2), lambda i: (0, i, 0)),
        out_shape=jax.ShapeDtypeStruct((2, cnt, H // 2), jnp.float32),
    )(fsd, fss, ea, we, b)


def _stats_body(h_ref, agg_ref, r_ref, st_ref):
    agg = jnp.concatenate([agg_ref[0], agg_ref[1]], axis=1)
    r = jnp.maximum(h_ref[...] + agg, 0.0)
    r_ref[...] = r

    @pl.when(pl.program_id(0) == 0)
    def _():
        st_ref[...] = jnp.zeros_like(st_ref)

    s = jnp.sum(r, axis=0)[None]
    s2 = jnp.sum(r * r, axis=0)[None]
    pad = jnp.zeros((6, H), jnp.float32)
    st_ref[...] = st_ref[...] + jnp.concatenate([s, s2, pad], axis=0)


def _stats(h, agg):
    return pl.pallas_call(
        _stats_body,
        grid=(N // NB,),
        in_specs=[
            pl.BlockSpec((NB, H), lambda i: (i, 0)),
            pl.BlockSpec((2, NB, H // 2), lambda i: (0, i, 0)),
        ],
        out_specs=[
            pl.BlockSpec((NB, H), lambda i: (i, 0)),
            pl.BlockSpec((8, H), lambda i: (0, 0)),
        ],
        out_shape=[
            jax.ShapeDtypeStruct((N, H), jnp.float32),
            jax.ShapeDtypeStruct((8, H), jnp.float32),
        ],
    )(h, agg)


def _bn(r, st_ref, g_ref, b_ref):
    mu = st_ref[0:1, :] / N
    var = st_ref[1:2, :] / N - mu * mu
    rstd = lax.rsqrt(var + 1e-5)
    return (r - mu) * rstd * g_ref[...] + b_ref[...]


def _norm_tables_body(r_ref, st_ref, g_ref, b_ref, wd_ref, ws_ref,
                      h_ref, td_ref, ts_ref):
    hn = _bn(r_ref[...], st_ref, g_ref, b_ref)
    h_ref[...] = hn
    td = jnp.dot(hn, wd_ref[...], preferred_element_type=jnp.float32)
    ts = jnp.dot(hn, ws_ref[...], preferred_element_type=jnp.float32)
    td_ref[...] = _pack_tables(td)
    ts_ref[...] = _pack_tables(ts)


def _norm_tables(r, st, g, b, wd, wsr):
    return pl.pallas_call(
        _norm_tables_body,
        grid=(N // NB,),
        in_specs=[
            pl.BlockSpec((NB, H), lambda i: (i, 0)),
            pl.BlockSpec((8, H), lambda i: (0, 0)),
            pl.BlockSpec((1, H), lambda i: (0, 0)),
            pl.BlockSpec((1, H), lambda i: (0, 0)),
            pl.BlockSpec((H, 2 * H), lambda i: (0, 0)),
            pl.BlockSpec((H, 2 * H), lambda i: (0, 0)),
        ],
        out_specs=[
            pl.BlockSpec((NB, H), lambda i: (i, 0)),
            pl.BlockSpec((NB, H), lambda i: (i, 0)),
            pl.BlockSpec((NB, H), lambda i: (i, 0)),
        ],
        out_shape=[
            jax.ShapeDtypeStruct((N, H), jnp.float32),
            jax.ShapeDtypeStruct((N, H), jnp.uint32),
            jax.ShapeDtypeStruct((N, H), jnp.uint32),
        ],
    )(r, st, g, b, wd, wsr)


def _final_body(r_ref, st_ref, g_ref, b_ref, bat_ref, w1_ref, b1_ref,
                w2_ref, b2_ref, out_ref):
    hn = _bn(r_ref[...], st_ref, g_ref, b_ref)
    gids = lax.broadcasted_iota(jnp.int32, (1, G), 1)
    onehot = (bat_ref[...] == gids).astype(jnp.float32)
    sums = lax.dot_general(onehot, hn, (((0,), (0,)), ((), ())),
                           preferred_element_type=jnp.float32)
    counts = jnp.sum(onehot, axis=0)[:, None]
    gemb = sums / jnp.maximum(counts, 1.0)
    o = jnp.maximum(jnp.dot(gemb, w1_ref[...],
                            preferred_element_type=jnp.float32) + b1_ref[...],
                    0.0)
    out_ref[...] = jnp.dot(o, w2_ref[...],
                           preferred_element_type=jnp.float32) + b2_ref[...]


def _final(r, st, g, b, batch2d, w1, b1, w2, b2):
    return pl.pallas_call(
        _final_body,
        out_shape=jax.ShapeDtypeStruct((G, 1), jnp.float32),
    )(r, st, g, b, batch2d, w1, b1, w2, b2)


KG = 40


def _sc_gather2(td, tsrc, dst, src, e_off, cnt):
    ep = cnt // NWORK
    nch = ep // KG
    nit = nch // 2
    mesh = plsc.VectorSubcoreMesh(core_axis_name="c", subcore_axis_name="s")

    @functools.partial(
        pl.kernel, mesh=mesh,
        out_type=[jax.ShapeDtypeStruct((cnt, H), jnp.uint32),
                  jax.ShapeDtypeStruct((cnt, H), jnp.uint32)],
        scratch_types=[
            pltpu.VMEM((KG,), jnp.int32),
            pltpu.VMEM((KG,), jnp.int32),
            pltpu.VMEM((KG,), jnp.int32),
            pltpu.VMEM((KG,), jnp.int32),
            pltpu.VMEM((KG, H), jnp.uint32),
            pltpu.VMEM((KG, H), jnp.uint32),
            pltpu.VMEM((KG, H), jnp.uint32),
            pltpu.VMEM((KG, H), jnp.uint32),
            pltpu.SemaphoreType.DMA,
            pltpu.SemaphoreType.DMA,
            pltpu.SemaphoreType.DMA,
            pltpu.SemaphoreType.DMA,
            pltpu.SemaphoreType.DMA,
        ],
    )
    def k(td_h, ts_h, dst_h, src_h, fsd_h, fss_h,
          di0, si0, di1, si1, rd0, rs0, rd1, rs1, sd0, ss0, sd1, ss1, sw):
        wid = lax.axis_index("s") * NCORES + lax.axis_index("c")
        base = wid * ep

        def gath(j, di, si, rd, rs, s_d, s_s):
            e0 = base + j * KG
            pltpu.sync_copy(dst_h.at[pl.ds(e0 + e_off, KG)], di)
            pltpu.sync_copy(src_h.at[pl.ds(e0 + e_off, KG)], si)
            pltpu.async_copy(td_h.at[di], rd, s_d)
            pltpu.async_copy(ts_h.at[si], rs, s_s)

        def waitg(di, si, rd, rs, s_d, s_s):
            pltpu.make_async_copy(td_h.at[di], rd, s_d).wait()
            pltpu.make_async_copy(ts_h.at[si], rs, s_s).wait()

        def put(j, rd, rs):
            pltpu.async_copy(rd, fsd_h.at[pl.ds(base + j * KG, KG)], sw)
            pltpu.async_copy(rs, fss_h.at[pl.ds(base + j * KG, KG)], sw)

        def waitw(rd, rs):
            pltpu.make_async_copy(rd, fsd_h.at[pl.ds(base, KG)], sw).wait()
            pltpu.make_async_copy(rs, fss_h.at[pl.ds(base, KG)], sw).wait()

        gath(0, di0, si0, rd0, rs0, sd0, ss0)

        def body(i, carry):
            j0 = 2 * i
            j1 = j0 + 1

            @pl.when(i > 0)
            def _():
                waitw(rd1, rs1)

            gath(j1, di1, si1, rd1, rs1, sd1, ss1)
            waitg(di0, si0, rd0, rs0, sd0, ss0)
            put(j0, rd0, rs0)

            @pl.when(i < nit - 1)
            def _():
                waitw(rd0, rs0)
                gath(j0 + 2, di0, si0, rd0, rs0, sd0, ss0)

            waitg(di1, si1, rd1, rs1, sd1, ss1)
            put(j1, rd1, rs1)
            return carry

        lax.fori_loop(0, nit, body, 0)
        waitw(rd0, rs0)
        waitw(rd1, rs1)

    return k(td, tsrc, dst, src)


def _sc_scatter_add(m, dst, e_off, cnt, init=None):
    tpc = cnt // NSUB
    nch = tpc // K
    rows_per_tile = NPAD // NSUB
    hc = H // 2
    mesh = plsc.VectorSubcoreMesh(core_axis_name="c", subcore_axis_name="s")

    nit = nch // 2

    @functools.partial(
        pl.kernel, mesh=mesh,
        out_type=jax.ShapeDtypeStruct((2, NPAD, hc), jnp.float32),
        scratch_types=[
            pltpu.VMEM((K,), jnp.int32),
            pltpu.VMEM((K,), jnp.int32),
            pltpu.VMEM((K, hc), jnp.float32),
            pltpu.VMEM((K, hc), jnp.float32),
            pltpu.VMEM((K, hc), jnp.float32),
            pltpu.VMEM_SHARED((NPAD, hc), jnp.float32),
            pltpu.SemaphoreType.DMA,
            pltpu.SemaphoreType.DMA,
        ],
    )
    def k(m_h, dst_h, *args):
        if init is None:
            agg_h, di0, di1, mv0, mv1, zv, acc, s0, s1 = args
        else:
            init_h, agg_h, di0, di1, mv0, mv1, zv, acc, s0, s1 = args
        c = lax.axis_index("c")
        s = lax.axis_index("s")
        r0 = s * rows_per_tile

        if init is None:
            def zrow(e, carry):
                for cc in range(hc // 16):
                    zv[e, pl.ds(cc * 16, 16)] = jnp.zeros((16,), jnp.float32)
                return carry

            lax.fori_loop(0, K, zrow, 0)

            def zchunk(j, carry):
                pltpu.sync_copy(zv, acc.at[pl.ds(r0 + j * K, K)])
                return carry

            lax.fori_loop(0, rows_per_tile // K, zchunk, 0)
        else:
            pltpu.sync_copy(init_h.at[c, pl.ds(r0, rows_per_tile)],
                            acc.at[pl.ds(r0, rows_per_tile)])

        plsc.subcore_barrier()

        base = s * tpc

        def getm(j, di, mv, sem):
            e0 = base + j * K
            pltpu.sync_copy(dst_h.at[pl.ds(e0 + e_off, K)], di)
            pltpu.async_copy(m_h.at[c, pl.ds(e0, K)], mv, sem)

        def waitm(mv, sem):
            pltpu.make_async_copy(m_h.at[c, pl.ds(base, K)], mv, sem).wait()

        getm(0, di0, mv0, s0)

        def body(i, carry):
            j0 = 2 * i
            j1 = j0 + 1
            getm(j1, di1, mv1, s1)
            waitm(mv0, s0)
            pltpu.sync_copy(mv0, acc.at[di0], add=True)

            @pl.when(i < nit - 1)
            def _():
                getm(j0 + 2, di0, mv0, s0)

            waitm(mv1, s1)
            pltpu.sync_copy(mv1, acc.at[di1], add=True)
            return carry

        lax.fori_loop(0, nit, body, 0)
        plsc.subcore_barrier()
        pltpu.sync_copy(acc.at[pl.ds(r0, rows_per_tile)],
                        agg_h.at[c, pl.ds(r0, rows_per_tile)])

    if init is None:
        return k(m, dst)
    return k(m, dst, init)


def kernel(x, edge_index, edge_attr, batch, W_emb, b_emb, Wf, bf, Ws, bs,
           gamma, beta, W1, b1, W2, b2):
    src = edge_index[0]
    dst = edge_index[1]

    wd = [jnp.concatenate([Wf[i, :H], Ws[i, :H]], axis=1) for i in range(L)]
    wsr = [jnp.concatenate([Wf[i, H:2 * H], Ws[i, H:2 * H]], axis=1)
           for i in range(L)]
    we = [jnp.concatenate([Wf[i, 2 * H:], Ws[i, 2 * H:]], axis=1)
          for i in range(L)]
    bfs = [jnp.concatenate([bf[i], bs[i]])[None] for i in range(L)]

    h, td, tsrc = _emb_tables(x, W_emb, b_emb[None], wd[0], wsr[0])

    for i in range(L):
        fsd0, fss0 = _sc_gather2(td, tsrc, dst, src, 0, HALF0)
        m0 = _edge_msg(fsd0, fss0, edge_attr, we[i], bfs[i], 0, HALF0)
        fsd1, fss1 = _sc_gather2(td, tsrc, dst, src, HALF0, HALF1)
        m1 = _edge_msg(fsd1, fss1, edge_attr, we[i], bfs[i], HALF0, HALF1)
        agg0 = _sc_scatter_add(m0, dst, 0, HALF0)
        agg = _sc_scatter_add(m1, dst, HALF0, HALF1, init=agg0)[:, :N, :]
        r, st = _stats(h, agg)
        if i < L - 1:
            h, td, tsrc = _norm_tables(r, st, gamma[i][None], beta[i][None],
                                       wd[i + 1], wsr[i + 1])
        else:
            pred = _final(r, st, gamma[i][None], beta[i][None],
                          batch[:, None], W1, b1[None], W2, b2[None])
    return pred.reshape((G,))
